# baseline (speedup 1.0000x reference)
import functools

import jax
import jax.numpy as jnp
from jax import lax
from jax.experimental import pallas as pl
from jax.experimental.pallas import tpu as pltpu
from jax.experimental.pallas import tpu_sc as plsc

_NODES = 100000
_G = 64
_D = 128
_BLK = 10000
_NBLK = _NODES // _BLK
_LOG2 = 0.6931471805599453

_NW = 32
_WROWS = 3128
_CH = 184
_NCH = _WROWS // _CH
_LAST_BASE = 31 * _WROWS
_LAST_FULL = 16
_LAST_TAIL = 3032 - _LAST_FULL * _CH
_TAIL_START = _LAST_BASE + _LAST_FULL * _CH


def _sc_segment_sums(b1, b2, z1, z2):
    mesh = plsc.VectorSubcoreMesh(core_axis_name="c", subcore_axis_name="s")

    @functools.partial(
        pl.kernel,
        mesh=mesh,
        out_type=[
            jax.ShapeDtypeStruct((_NW, _G, _D), jnp.float32),
            jax.ShapeDtypeStruct((_NW, _G, _D), jnp.float32),
        ],
        scratch_types=[
            pltpu.VMEM((_CH, _D), jnp.float32),
            pltpu.VMEM((_CH, _D), jnp.float32),
            pltpu.VMEM((_G, _D), jnp.float32),
            pltpu.VMEM((_G, _D), jnp.float32),
            pltpu.VMEM((_WROWS,), jnp.int32),
            pltpu.VMEM((_WROWS,), jnp.int32),
            pltpu.SemaphoreType.DMA,
        ],
        compiler_params=pltpu.CompilerParams(needs_layout_passes=False),
    )
    def seg(b1_hbm, b2_hbm, z1_hbm, z2_hbm, out1_hbm, out2_hbm,
            zb1, zb2, acc1, acc2, bv1, bv2, dsem):
        wid = lax.axis_index("s") * 2 + lax.axis_index("c")
        base = wid * _WROWS

        zeros16 = jnp.zeros((16,), jnp.float32)
        lane = lax.iota(jnp.int32, 16)

        for r in range(_G):
            for j in range(_D // 16):
                acc1[r, pl.ds(j * 16, 16)] = zeros16
                acc2[r, pl.ds(j * 16, 16)] = zeros16

        @pl.when(wid != _NW - 1)
        def _load_b_full():
            c3 = pltpu.async_copy(b1_hbm.at[pl.ds(base, _WROWS)], bv1, dsem)
            c4 = pltpu.async_copy(b2_hbm.at[pl.ds(base, _WROWS)], bv2, dsem)
            c3.wait()
            c4.wait()

        @pl.when(wid == _NW - 1)
        def _load_b_tail():
            nlast = 3032
            c3 = pltpu.async_copy(b1_hbm.at[pl.ds(_LAST_BASE, nlast)],
                                  bv1.at[pl.ds(0, nlast)], dsem)
            c4 = pltpu.async_copy(b2_hbm.at[pl.ds(_LAST_BASE, nlast)],
                                  bv2.at[pl.ds(0, nlast)], dsem)
            c3.wait()
            c4.wait()

        dn = lax.GatherDimensionNumbers(
            offset_dims=(), collapsed_slice_dims=(0,),
            start_index_map=(0,))

        def bcast(bv, r):
            w16 = pl.multiple_of((r // 16) * 16, 16)
            pos = jnp.full((16, 1), r % 16, jnp.int32)
            return lax.gather(bv[pl.ds(w16, 16)], pos, dn, (1,),
                              mode=lax.GatherScatterMode.PROMISE_IN_BOUNDS)

        def do_chunk(start, coff, nrows, crows=_CH):
            c1 = pltpu.async_copy(z1_hbm.at[pl.ds(start, crows)],
                                  zb1.at[pl.ds(0, crows)], dsem)
            c2 = pltpu.async_copy(z2_hbm.at[pl.ds(start, crows)],
                                  zb2.at[pl.ds(0, crows)], dsem)
            c1.wait()
            c2.wait()

            b1_first = bcast(bv1, coff)
            b1_last = bcast(bv1, coff + crows - 1)
            b2_first = bcast(bv2, coff)
            b2_last = bcast(bv2, coff + crows - 1)
            uni = (lax.reduce_min((b1_first == b1_last).astype(jnp.int32), (0,))
                   * lax.reduce_min((b2_first == b2_last).astype(jnp.int32),
                                    (0,)))

            @pl.when(uni == 1)
            def _fast():
                NJ = _D // 16
                init = tuple(
                    [plsc.load_gather(acc1, [b1_first, lane + j * 16])
                     for j in range(NJ)]
                    + [plsc.load_gather(acc2, [b2_first, lane + j * 16])
                       for j in range(NJ)])

                def row(r, regs):
                    o1 = tuple(regs[j] + zb1[r, pl.ds(j * 16, 16)]
                               for j in range(NJ))
                    o2 = tuple(regs[NJ + j] + zb2[r, pl.ds(j * 16, 16)]
                               for j in range(NJ))
                    return o1 + o2

                regs = lax.fori_loop(0, nrows, row, init)
                for j in range(NJ):
                    plsc.store_scatter(acc1, [b1_first, lane + j * 16],
                                       regs[j])
                    plsc.store_scatter(acc2, [b2_first, lane + j * 16],
                                       regs[NJ + j])

            @pl.when(uni == 0)
            def _slow():
                def row(r, _):
                    b1v = bcast(bv1, coff + r)
                    b2v = bcast(bv2, coff + r)
                    for j in range(_D // 16):
                        dvec = lane + (j * 16)
                        sl = pl.ds(j * 16, 16)
                        plsc.addupdate_scatter(acc1, [b1v, dvec], zb1[r, sl])
                        plsc.addupdate_scatter(acc2, [b2v, dvec], zb2[r, sl])
                    return 0

                lax.fori_loop(0, nrows, row, 0)

        nch = jnp.where(wid == _NW - 1, _LAST_FULL, _NCH)

        def chunk_loop(c, _):
            do_chunk(base + c * _CH, c * _CH, _CH)
            return 0

        lax.fori_loop(0, nch, chunk_loop, 0)

        @pl.when(wid == _NW - 1)
        def _tail():
            do_chunk(_TAIL_START, _LAST_FULL * _CH, _LAST_TAIL,
                     crows=_LAST_TAIL)

        pltpu.sync_copy(acc1, out1_hbm.at[wid])
        pltpu.sync_copy(acc2, out2_hbm.at[wid])

    return seg(b1, b2, z1, z2)


def _ep_sum(t):
    a = -t
    sp = jnp.maximum(a, 0.0) + jnp.log1p(jnp.exp(-jnp.abs(a)))
    return jnp.sum(_LOG2 - sp)


def _tc_body(b1_ref, b2_ref, z1_ref, z2_ref, p1_ref, p2_ref, out_ref,
             g1_ref, g2_ref, acc_ref, comp_ref):
    i = pl.program_id(0)

    b1 = b1_ref[0]
    b2 = b2_ref[0]

    @pl.when(i == 0)
    def _init():
        g1_ref[...] = jnp.sum(p1_ref[...], axis=0)
        g2_ref[...] = jnp.sum(p2_ref[...], axis=0)
        out_ref[...] = jnp.zeros_like(out_ref)
        acc_ref[...] = jnp.zeros_like(acc_ref)
        comp_ref[...] = jnp.zeros_like(comp_ref)

    gcat = jnp.concatenate([g1_ref[...], g2_ref[...]],
                           axis=0).astype(jnp.bfloat16)
    z1b = z1_ref[...].astype(jnp.bfloat16)
    z2b = z2_ref[...].astype(jnp.bfloat16)
    s1 = lax.dot_general(z1b, gcat, (((1,), (1,)), ((), ())),
                         preferred_element_type=jnp.float32)
    s2 = lax.dot_general(z2b, gcat, (((1,), (1,)), ((), ())),
                         preferred_element_type=jnp.float32)
    iota_n = lax.broadcasted_iota(jnp.int32, (_BLK, _G), 1)
    oh1 = (iota_n == jnp.broadcast_to(b1.T, (_BLK, _G))).astype(jnp.float32)
    oh2 = (iota_n == jnp.broadcast_to(b2.T, (_BLK, _G))).astype(jnp.float32)
    t11 = jnp.sum(s1[:, :_G] * oh1, axis=1)
    t12 = jnp.sum(s1[:, _G:] * oh1, axis=1)
    t21 = jnp.sum(s2[:, :_G] * oh2, axis=1)
    t22 = jnp.sum(s2[:, _G:] * oh2, axis=1)
    blk = jnp.concatenate([
        jnp.reshape(_ep_sum(t11), (1, 1)),
        jnp.reshape(_ep_sum(t12), (1, 1)),
        jnp.reshape(_ep_sum(t22), (1, 1)),
        jnp.reshape(_ep_sum(t21), (1, 1)),
    ], axis=1)
    y = blk - comp_ref[...]
    t = acc_ref[...] + y
    comp_ref[...] = (t - acc_ref[...]) - y
    acc_ref[...] = t

    @pl.when(i == _NBLK - 1)
    def _finish():
        total = acc_ref[...] - comp_ref[...]
        a11 = total[0:1, 0:1]
        a12 = total[0:1, 1:2]
        a22 = total[0:1, 2:3]
        a21 = total[0:1, 3:4]
        out_ref[...] = (a11 - a12) - (a22 - a21)


@jax.jit
def kernel(batch_1, batch_2, z1, z2):
    b1i = batch_1.astype(jnp.int32)
    b2i = batch_2.astype(jnp.int32)
    parts1, parts2 = _sc_segment_sums(b1i, b2i, z1, z2)
    b1r = b1i.reshape(_NBLK, 1, _BLK)
    b2r = b2i.reshape(_NBLK, 1, _BLK)
    out = pl.pallas_call(
        _tc_body,
        grid=(_NBLK,),
        in_specs=[
            pl.BlockSpec((1, 1, _BLK), lambda i: (i, 0, 0)),
            pl.BlockSpec((1, 1, _BLK), lambda i: (i, 0, 0)),
            pl.BlockSpec((_BLK, _D), lambda i: (i, 0)),
            pl.BlockSpec((_BLK, _D), lambda i: (i, 0)),
            pl.BlockSpec((_NW, _G, _D), lambda i: (0, 0, 0)),
            pl.BlockSpec((_NW, _G, _D), lambda i: (0, 0, 0)),
        ],
        out_specs=pl.BlockSpec((1, 1), lambda i: (0, 0)),
        out_shape=jax.ShapeDtypeStruct((1, 1), jnp.float32),
        scratch_shapes=[
            pltpu.VMEM((_G, _D), jnp.float32),
            pltpu.VMEM((_G, _D), jnp.float32),
            pltpu.VMEM((1, 4), jnp.float32),
            pltpu.VMEM((1, 4), jnp.float32),
        ],
        compiler_params=pltpu.CompilerParams(
            dimension_semantics=("arbitrary",),
        ),
    )(b1r, b2r, z1, z2, parts1, parts2)
    return out[0, 0]


# speedup vs baseline: 1.0138x; 1.0138x over previous
import functools

import jax
import jax.numpy as jnp
from jax import lax
from jax.experimental import pallas as pl
from jax.experimental.pallas import tpu as pltpu
from jax.experimental.pallas import tpu_sc as plsc

_NODES = 100000
_G = 64
_D = 128
_BLK = 10000
_NBLK = _NODES // _BLK
_LOG2 = 0.6931471805599453

_NW = 32
_WROWS = 3128
_CH = 136
_NCH = _WROWS // _CH
_LAST_BASE = 31 * _WROWS
_LAST_FULL = 22
_LAST_TAIL = 3032 - _LAST_FULL * _CH
_TAIL_START = _LAST_BASE + _LAST_FULL * _CH


def _sc_segment_sums(b1, b2, z1, z2):
    mesh = plsc.VectorSubcoreMesh(core_axis_name="c", subcore_axis_name="s")

    @functools.partial(
        pl.kernel,
        mesh=mesh,
        out_type=[
            jax.ShapeDtypeStruct((_NW, _G, _D), jnp.float32),
            jax.ShapeDtypeStruct((_NW, _G, _D), jnp.float32),
        ],
        scratch_types=[
            pltpu.VMEM((_CH, _D), jnp.float32),
            pltpu.VMEM((_CH, _D), jnp.float32),
            pltpu.VMEM((_G, _D), jnp.float32),
            pltpu.VMEM((_G, _D), jnp.float32),
            pltpu.VMEM((_WROWS,), jnp.int32),
            pltpu.VMEM((_WROWS,), jnp.int32),
            pltpu.SemaphoreType.DMA,
        ],
        compiler_params=pltpu.CompilerParams(needs_layout_passes=False),
    )
    def seg(b1_hbm, b2_hbm, z1_hbm, z2_hbm, out1_hbm, out2_hbm,
            zb1, zb2, acc1, acc2, bv1, bv2, dsem):
        wid = lax.axis_index("s") * 2 + lax.axis_index("c")
        base = wid * _WROWS

        zeros16 = jnp.zeros((16,), jnp.float32)
        lane = lax.iota(jnp.int32, 16)

        for r in range(_G):
            for j in range(_D // 16):
                acc1[r, pl.ds(j * 16, 16)] = zeros16
                acc2[r, pl.ds(j * 16, 16)] = zeros16

        @pl.when(wid != _NW - 1)
        def _load_b_full():
            c3 = pltpu.async_copy(b1_hbm.at[pl.ds(base, _WROWS)], bv1, dsem)
            c4 = pltpu.async_copy(b2_hbm.at[pl.ds(base, _WROWS)], bv2, dsem)
            c3.wait()
            c4.wait()

        @pl.when(wid == _NW - 1)
        def _load_b_tail():
            nlast = 3032
            c3 = pltpu.async_copy(b1_hbm.at[pl.ds(_LAST_BASE, nlast)],
                                  bv1.at[pl.ds(0, nlast)], dsem)
            c4 = pltpu.async_copy(b2_hbm.at[pl.ds(_LAST_BASE, nlast)],
                                  bv2.at[pl.ds(0, nlast)], dsem)
            c3.wait()
            c4.wait()

        dn = lax.GatherDimensionNumbers(
            offset_dims=(), collapsed_slice_dims=(0,),
            start_index_map=(0,))

        def bcast(bv, r):
            w16 = pl.multiple_of((r // 16) * 16, 16)
            pos = jnp.full((16, 1), r % 16, jnp.int32)
            return lax.gather(bv[pl.ds(w16, 16)], pos, dn, (1,),
                              mode=lax.GatherScatterMode.PROMISE_IN_BOUNDS)

        def do_chunk(start, coff, nrows, crows=_CH):
            c1 = pltpu.async_copy(z1_hbm.at[pl.ds(start, crows)],
                                  zb1.at[pl.ds(0, crows)], dsem)
            c2 = pltpu.async_copy(z2_hbm.at[pl.ds(start, crows)],
                                  zb2.at[pl.ds(0, crows)], dsem)
            c1.wait()
            c2.wait()

            b1_first = bcast(bv1, coff)
            b1_last = bcast(bv1, coff + crows - 1)
            b2_first = bcast(bv2, coff)
            b2_last = bcast(bv2, coff + crows - 1)
            uni = (lax.reduce_min((b1_first == b1_last).astype(jnp.int32), (0,))
                   * lax.reduce_min((b2_first == b2_last).astype(jnp.int32),
                                    (0,)))

            @pl.when(uni == 1)
            def _fast():
                NJ = _D // 16
                init = tuple(
                    [plsc.load_gather(acc1, [b1_first, lane + j * 16])
                     for j in range(NJ)]
                    + [plsc.load_gather(acc2, [b2_first, lane + j * 16])
                       for j in range(NJ)])

                def row(r, regs):
                    o1 = tuple(regs[j] + zb1[r, pl.ds(j * 16, 16)]
                               for j in range(NJ))
                    o2 = tuple(regs[NJ + j] + zb2[r, pl.ds(j * 16, 16)]
                               for j in range(NJ))
                    return o1 + o2

                regs = lax.fori_loop(0, nrows, row, init)
                for j in range(NJ):
                    plsc.store_scatter(acc1, [b1_first, lane + j * 16],
                                       regs[j])
                    plsc.store_scatter(acc2, [b2_first, lane + j * 16],
                                       regs[NJ + j])

            @pl.when(uni == 0)
            def _slow():
                def row(r, _):
                    b1v = bcast(bv1, coff + r)
                    b2v = bcast(bv2, coff + r)
                    for j in range(_D // 16):
                        dvec = lane + (j * 16)
                        sl = pl.ds(j * 16, 16)
                        plsc.addupdate_scatter(acc1, [b1v, dvec], zb1[r, sl])
                        plsc.addupdate_scatter(acc2, [b2v, dvec], zb2[r, sl])
                    return 0

                lax.fori_loop(0, nrows, row, 0)

        nch = jnp.where(wid == _NW - 1, _LAST_FULL, _NCH)

        def chunk_loop(c, _):
            do_chunk(base + c * _CH, c * _CH, _CH)
            return 0

        lax.fori_loop(0, nch, chunk_loop, 0)

        @pl.when(wid == _NW - 1)
        def _tail():
            do_chunk(_TAIL_START, _LAST_FULL * _CH, _LAST_TAIL,
                     crows=_LAST_TAIL)

        pltpu.sync_copy(acc1, out1_hbm.at[wid])
        pltpu.sync_copy(acc2, out2_hbm.at[wid])

    return seg(b1, b2, z1, z2)


def _ep_sum(t):
    a = -t
    sp = jnp.maximum(a, 0.0) + jnp.log1p(jnp.exp(-jnp.abs(a)))
    return jnp.sum(_LOG2 - sp)


def _tc_body(b1_ref, b2_ref, z1_ref, z2_ref, p1_ref, p2_ref, out_ref,
             g1_ref, g2_ref, acc_ref, comp_ref):
    i = pl.program_id(0)

    b1 = b1_ref[0]
    b2 = b2_ref[0]

    @pl.when(i == 0)
    def _init():
        g1_ref[...] = jnp.sum(p1_ref[...], axis=0)
        g2_ref[...] = jnp.sum(p2_ref[...], axis=0)
        out_ref[...] = jnp.zeros_like(out_ref)
        acc_ref[...] = jnp.zeros_like(acc_ref)
        comp_ref[...] = jnp.zeros_like(comp_ref)

    gcat = jnp.concatenate([g1_ref[...], g2_ref[...]],
                           axis=0).astype(jnp.bfloat16)
    z1b = z1_ref[...].astype(jnp.bfloat16)
    z2b = z2_ref[...].astype(jnp.bfloat16)
    s1 = lax.dot_general(z1b, gcat, (((1,), (1,)), ((), ())),
                         preferred_element_type=jnp.float32)
    s2 = lax.dot_general(z2b, gcat, (((1,), (1,)), ((), ())),
                         preferred_element_type=jnp.float32)
    iota_n = lax.broadcasted_iota(jnp.int32, (_BLK, _G), 1)
    oh1 = (iota_n == jnp.broadcast_to(b1.T, (_BLK, _G))).astype(jnp.float32)
    oh2 = (iota_n == jnp.broadcast_to(b2.T, (_BLK, _G))).astype(jnp.float32)
    t11 = jnp.sum(s1[:, :_G] * oh1, axis=1)
    t12 = jnp.sum(s1[:, _G:] * oh1, axis=1)
    t21 = jnp.sum(s2[:, :_G] * oh2, axis=1)
    t22 = jnp.sum(s2[:, _G:] * oh2, axis=1)
    blk = jnp.concatenate([
        jnp.reshape(_ep_sum(t11), (1, 1)),
        jnp.reshape(_ep_sum(t12), (1, 1)),
        jnp.reshape(_ep_sum(t22), (1, 1)),
        jnp.reshape(_ep_sum(t21), (1, 1)),
    ], axis=1)
    y = blk - comp_ref[...]
    t = acc_ref[...] + y
    comp_ref[...] = (t - acc_ref[...]) - y
    acc_ref[...] = t

    @pl.when(i == _NBLK - 1)
    def _finish():
        total = acc_ref[...] - comp_ref[...]
        a11 = total[0:1, 0:1]
        a12 = total[0:1, 1:2]
        a22 = total[0:1, 2:3]
        a21 = total[0:1, 3:4]
        out_ref[...] = (a11 - a12) - (a22 - a21)


@jax.jit
def kernel(batch_1, batch_2, z1, z2):
    b1i = batch_1.astype(jnp.int32)
    b2i = batch_2.astype(jnp.int32)
    parts1, parts2 = _sc_segment_sums(b1i, b2i, z1, z2)
    b1r = b1i.reshape(_NBLK, 1, _BLK)
    b2r = b2i.reshape(_NBLK, 1, _BLK)
    out = pl.pallas_call(
        _tc_body,
        grid=(_NBLK,),
        in_specs=[
            pl.BlockSpec((1, 1, _BLK), lambda i: (i, 0, 0)),
            pl.BlockSpec((1, 1, _BLK), lambda i: (i, 0, 0)),
            pl.BlockSpec((_BLK, _D), lambda i: (i, 0)),
            pl.BlockSpec((_BLK, _D), lambda i: (i, 0)),
            pl.BlockSpec((_NW, _G, _D), lambda i: (0, 0, 0)),
            pl.BlockSpec((_NW, _G, _D), lambda i: (0, 0, 0)),
        ],
        out_specs=pl.BlockSpec((1, 1), lambda i: (0, 0)),
        out_shape=jax.ShapeDtypeStruct((1, 1), jnp.float32),
        scratch_shapes=[
            pltpu.VMEM((_G, _D), jnp.float32),
            pltpu.VMEM((_G, _D), jnp.float32),
            pltpu.VMEM((1, 4), jnp.float32),
            pltpu.VMEM((1, 4), jnp.float32),
        ],
        compiler_params=pltpu.CompilerParams(
            dimension_semantics=("arbitrary",),
        ),
    )(b1r, b2r, z1, z2, parts1, parts2)
    return out[0, 0]
